# Initial kernel scaffold; baseline (speedup 1.0000x reference)
import jax
import jax.numpy as jnp
from jax.experimental import pallas as pl


def kernel(z_e, embeddings, logit_scales):
    raise NotImplementedError("write your pallas kernel here")


# speedup vs baseline: 2.8791x; 2.8791x over previous
import functools

import jax
import jax.numpy as jnp
from jax.experimental import pallas as pl
from jax.experimental.pallas import tpu as pltpu

NH = 4
CODES = 1024
EMB = 1024
HD = EMB // NH
NTOK = 16384
BT = 2048


def _vq_kernel(scales_ref, z_ref, emb_ref, zq_ref, idx_ref, comb_ref, perp_ref,
               psum_ref):
    t = pl.program_id(0)
    h = pl.program_id(1)

    @pl.when(jnp.logical_and(t == 0, h == 0))
    def _init():
        psum_ref[...] = jnp.zeros_like(psum_ref)

    z = z_ref[...]
    emb = emb_ref[0]

    zn = z * jax.lax.rsqrt(jnp.maximum(jnp.sum(z * z, axis=-1, keepdims=True),
                                       1e-24))
    en = emb * jax.lax.rsqrt(
        jnp.maximum(jnp.sum(emb * emb, axis=-1, keepdims=True), 1e-24))

    scale = scales_ref[h]
    logits = scale * jax.lax.dot_general(
        zn, en, (((1,), (1,)), ((), ())),
        preferred_element_type=jnp.float32)

    m = jnp.max(logits, axis=-1, keepdims=True)
    iota = jax.lax.broadcasted_iota(jnp.int32, logits.shape, 1)
    idx = jnp.min(jnp.where(logits >= m, iota, CODES), axis=-1)

    probs = jnp.exp(logits - m)
    probs = probs / jnp.sum(probs, axis=-1, keepdims=True)
    psum_ref[h, :] = psum_ref[h, :] + jnp.sum(probs, axis=0)

    onehot = (iota == idx[:, None]).astype(jnp.float32)
    zq_ref[...] = jax.lax.dot_general(
        onehot, emb, (((1,), (0,)), ((), ())),
        preferred_element_type=jnp.float32)

    idx_ref[0, 0, :] = idx

    @pl.when(h == 0)
    def _comb0():
        comb_ref[0, 0, :] = idx

    @pl.when(h > 0)
    def _combh():
        comb_ref[0, 0, :] = comb_ref[0, 0, :] * CODES + idx

    @pl.when(jnp.logical_and(t == pl.num_programs(0) - 1, h == NH - 1))
    def _finish():
        p = psum_ref[...] * (1.0 / NTOK)
        ent = jnp.sum(p * jnp.log(p + 1e-10), axis=-1, keepdims=True)
        perp_ref[0, 0] = jnp.mean(jnp.exp(-ent))


@functools.partial(jax.jit, static_argnames=())
def kernel(z_e, embeddings, logit_scales):
    nt = NTOK // BT
    grid = (nt, NH)
    zq, idx, comb, perp = pl.pallas_call(
        _vq_kernel,
        grid=grid,
        in_specs=[
            pl.BlockSpec(memory_space=pltpu.SMEM),
            pl.BlockSpec((BT, HD), lambda t, h: (t, h)),
            pl.BlockSpec((1, CODES, HD), lambda t, h: (h, 0, 0)),
        ],
        out_specs=[
            pl.BlockSpec((BT, HD), lambda t, h: (t, h)),
            pl.BlockSpec((1, 1, BT), lambda t, h: (h, 0, t)),
            pl.BlockSpec((1, 1, BT), lambda t, h: (0, 0, t)),
            pl.BlockSpec((1, 1), lambda t, h: (0, 0),
                         memory_space=pltpu.SMEM),
        ],
        out_shape=[
            jax.ShapeDtypeStruct((NTOK, EMB), jnp.float32),
            jax.ShapeDtypeStruct((NH, 1, NTOK), jnp.int32),
            jax.ShapeDtypeStruct((1, 1, NTOK), jnp.int32),
            jax.ShapeDtypeStruct((1, 1), jnp.float32),
        ],
        scratch_shapes=[pltpu.VMEM((NH, CODES), jnp.float32)],
    )(logit_scales, z_e, embeddings)

    temperature = jnp.asarray(1.0, dtype=jnp.float32)
    commitment_loss = jnp.asarray(0.0, dtype=jnp.float32)
    return (zq, comb[0, 0], perp[0, 0], temperature, commitment_loss)


# speedup vs baseline: 3.6408x; 1.2646x over previous
import functools

import jax
import jax.numpy as jnp
from jax.experimental import pallas as pl
from jax.experimental.pallas import tpu as pltpu

NH = 4
CODES = 1024
EMB = 1024
HD = EMB // NH
NTOK = 16384
BT = 2048


def _vq_kernel(scales_ref, z_ref, emb_ref, niota_ref, zq_ref, idx_ref,
               comb_ref, perp_ref, psum_ref):
    t = pl.program_id(0)
    h = pl.program_id(1)

    @pl.when(jnp.logical_and(t == 0, h == 0))
    def _init():
        psum_ref[...] = jnp.zeros_like(psum_ref)

    z = z_ref[...]
    emb = emb_ref[0]

    zn = z * jax.lax.rsqrt(jnp.maximum(jnp.sum(z * z, axis=-1, keepdims=True),
                                       1e-24))
    en = emb * jax.lax.rsqrt(
        jnp.maximum(jnp.sum(emb * emb, axis=-1, keepdims=True), 1e-24))

    scale = scales_ref[h]
    logits = scale * jax.lax.dot_general(
        zn, en, (((1,), (1,)), ((), ())),
        preferred_element_type=jnp.float32)

    m = jnp.max(logits, axis=-1, keepdims=True)
    niota = niota_ref[...]

    cand = jnp.where(logits >= m, niota, -3.0e38)
    widx = jnp.max(cand, axis=-1, keepdims=True)
    idx = (-widx[:, 0]).astype(jnp.int32)

    onehot = (cand == widx).astype(jnp.float32)
    zq_ref[...] = jax.lax.dot_general(
        onehot, emb, (((1,), (0,)), ((), ())),
        preferred_element_type=jnp.float32)

    e = jnp.exp(logits - m)
    inv = 1.0 / jnp.sum(e, axis=-1, keepdims=True)
    colsum = jax.lax.dot_general(
        inv, e, (((0,), (0,)), ((), ())),
        preferred_element_type=jnp.float32)
    psum_ref[h, :] = psum_ref[h, :] + colsum[0]

    idx_ref[0, 0, :] = idx

    @pl.when(h == 0)
    def _comb0():
        comb_ref[0, 0, :] = idx

    @pl.when(h > 0)
    def _combh():
        comb_ref[0, 0, :] = comb_ref[0, 0, :] * CODES + idx

    @pl.when(jnp.logical_and(t == pl.num_programs(0) - 1, h == NH - 1))
    def _finish():
        p = psum_ref[...] * (1.0 / NTOK)
        ent = jnp.sum(p * jnp.log(p + 1e-10), axis=-1, keepdims=True)
        perp_ref[0, 0] = jnp.mean(jnp.exp(-ent))


@functools.partial(jax.jit, static_argnames=())
def kernel(z_e, embeddings, logit_scales):
    nt = NTOK // BT
    grid = (nt, NH)
    niota = -jnp.arange(CODES, dtype=jnp.float32).reshape(1, CODES)
    zq, idx, comb, perp = pl.pallas_call(
        _vq_kernel,
        grid=grid,
        in_specs=[
            pl.BlockSpec(memory_space=pltpu.SMEM),
            pl.BlockSpec((BT, HD), lambda t, h: (t, h)),
            pl.BlockSpec((1, CODES, HD), lambda t, h: (h, 0, 0)),
            pl.BlockSpec((1, CODES), lambda t, h: (0, 0)),
        ],
        out_specs=[
            pl.BlockSpec((BT, HD), lambda t, h: (t, h)),
            pl.BlockSpec((1, 1, BT), lambda t, h: (h, 0, t)),
            pl.BlockSpec((1, 1, BT), lambda t, h: (0, 0, t)),
            pl.BlockSpec((1, 1), lambda t, h: (0, 0),
                         memory_space=pltpu.SMEM),
        ],
        out_shape=[
            jax.ShapeDtypeStruct((NTOK, EMB), jnp.float32),
            jax.ShapeDtypeStruct((NH, 1, NTOK), jnp.int32),
            jax.ShapeDtypeStruct((1, 1, NTOK), jnp.int32),
            jax.ShapeDtypeStruct((1, 1), jnp.float32),
        ],
        scratch_shapes=[pltpu.VMEM((NH, CODES), jnp.float32)],
    )(logit_scales, z_e, embeddings, niota)

    temperature = jnp.asarray(1.0, dtype=jnp.float32)
    commitment_loss = jnp.asarray(0.0, dtype=jnp.float32)
    return (zq, comb[0, 0], perp[0, 0], temperature, commitment_loss)
